# baseline (speedup 1.0000x reference)
import functools

import jax
import jax.numpy as jnp
from jax import lax
from jax.experimental import pallas as pl
from jax.experimental.pallas import tpu as pltpu
from jax.experimental.pallas import tpu_sc as plsc

N = 10000
E = 320000
D = 128
HEADS = 4
DH = 512
NP = 10240
BLK = 1024
NEG = -1e30

NSC = 16
EB = 64
NBLK = 320
NSUP = NBLK // 4
NITER = NBLK // 8
E_TILE = NBLK * EB
E_PAD = NSC * E_TILE
STRIPE = NP // NSC


def _scal_body(ft_ref, u8_ref, o_ref):
    i = pl.program_id(0)
    t_chunks = ft_ref.shape[0]
    acc = jnp.zeros((8, BLK), jnp.float32)
    for t in range(t_chunks):
        acc += jax.lax.dot_general(
            u8_ref[t], ft_ref[t], (((1,), (1,)), ((), ())),
            preferred_element_type=jnp.float32)
    col = i * BLK + jax.lax.broadcasted_iota(jnp.int32, (8, BLK), 1)
    o_ref[...] = jnp.where(col < N, acc, NEG)


def _scalar_tables(ft, u8):
    t = ft.shape[0]
    return pl.pallas_call(
        _scal_body,
        out_shape=jax.ShapeDtypeStruct((8, NP), jnp.float32),
        grid=(NP // BLK,),
        in_specs=[
            pl.BlockSpec((t, BLK, D), lambda i: (0, i, 0)),
            pl.BlockSpec((t, 8, D), lambda i: (0, 0, 0)),
        ],
        out_specs=pl.BlockSpec((8, BLK), lambda i: (0, i)),
    )(ft, u8)


def _fin1_body(p_ref, h_ref, snd_ref, w_ref, ws_ref, o_ref):
    sn = snd_ref[:, 0:1]
    for i in range(HEADS):
        di = snd_ref[:, 1 + i:2 + i]
        pn = p_ref[i] / (di + 1e-9)
        agg = jax.lax.dot_general(pn, w_ref[i], (((1,), (0,)), ((), ())),
                                  preferred_element_type=jnp.float32)
        res = jax.lax.dot_general(h_ref[...], ws_ref[i], (((1,), (0,)), ((), ())),
                                  preferred_element_type=jnp.float32)
        o_ref[i] = jnp.maximum(agg * sn + res, 0.0)


def _finish1(P, h_t, snd, W1, W1_self):
    return pl.pallas_call(
        _fin1_body,
        out_shape=jax.ShapeDtypeStruct((HEADS, NP, D), jnp.float32),
        grid=(NP // BLK,),
        in_specs=[
            pl.BlockSpec((HEADS, BLK, D), lambda i: (0, i, 0)),
            pl.BlockSpec((BLK, D), lambda i: (i, 0)),
            pl.BlockSpec((BLK, 8), lambda i: (i, 0)),
            pl.BlockSpec((HEADS, D, D), lambda i: (0, 0, 0)),
            pl.BlockSpec((HEADS, D, D), lambda i: (0, 0, 0)),
        ],
        out_specs=pl.BlockSpec((HEADS, BLK, D), lambda i: (0, i, 0)),
    )(P, h_t, snd, W1, W1_self)


def _fin2_body(q_ref, h1_ref, snd_ref, w_ref, ws_ref, o_ref):
    sn = snd_ref[:, 0:1]
    dinv = 1.0 / (snd_ref[:, 1:2] + 1e-9)
    acc = jnp.zeros((BLK, DH), jnp.float32)
    res = jnp.zeros((BLK, DH), jnp.float32)
    for i in range(HEADS):
        acc += jax.lax.dot_general(q_ref[i] * dinv, w_ref[pl.ds(i * D, D)],
                                   (((1,), (0,)), ((), ())),
                                   preferred_element_type=jnp.float32)
        res += jax.lax.dot_general(h1_ref[i], ws_ref[pl.ds(i * D, D)],
                                   (((1,), (0,)), ((), ())),
                                   preferred_element_type=jnp.float32)
    o_ref[...] = jnp.maximum(acc * sn + res, 0.0)


def _finish2(Q, h1cols, snd2, W2, W2_self):
    return pl.pallas_call(
        _fin2_body,
        out_shape=jax.ShapeDtypeStruct((NP, DH), jnp.float32),
        grid=(NP // BLK,),
        in_specs=[
            pl.BlockSpec((HEADS, BLK, D), lambda i: (0, i, 0)),
            pl.BlockSpec((HEADS, BLK, D), lambda i: (0, i, 0)),
            pl.BlockSpec((BLK, 8), lambda i: (i, 0)),
            pl.BlockSpec((DH, DH), lambda i: (0, 0)),
            pl.BlockSpec((DH, DH), lambda i: (0, 0)),
        ],
        out_specs=pl.BlockSpec((BLK, DH), lambda i: (i, 0)),
    )(Q, h1cols, snd2, W2, W2_self)


def _bcast_lane(v, lane):
    idx = jnp.full((16, 1), lane, jnp.int32)
    dnums = lax.GatherDimensionNumbers(
        offset_dims=(), collapsed_slice_dims=(0,), start_index_map=(0,))
    return lax.gather(v, idx, dnums, (1,),
                      mode=lax.GatherScatterMode.PROMISE_IN_BOUNDS)


def _make_sc_edge(n_tables):
    l2 = n_tables == 4

    scratch = [
        pltpu.VMEM((4, EB), jnp.int32),
        pltpu.VMEM((4, EB), jnp.int32),
        pltpu.VMEM((4, EB), jnp.int32),
        pltpu.VMEM((4, EB), jnp.int32),
    ]
    scratch += [pltpu.VMEM((EB,), jnp.int32)] * 4
    scratch += [pltpu.VMEM((EB,), jnp.int32)] * 4
    scratch += [pltpu.VMEM((EB,), jnp.int32)] * 4
    scratch += [pltpu.VMEM((EB,), jnp.int32)] * 4
    scratch += [pltpu.VMEM((EB,), jnp.float32)] * 4
    scratch += [pltpu.VMEM((EB,), jnp.float32)] * 4
    scratch += [pltpu.VMEM((EB,), jnp.float32)] * 4
    scratch += [pltpu.VMEM((EB, D), jnp.float32)] * 4
    scratch += [
        pltpu.VMEM_SHARED((NP, D), jnp.float32),
        pltpu.VMEM_SHARED((NP,), jnp.float32),
        pltpu.VMEM_SHARED((4 * NP,), jnp.float32),
    ]
    scratch += [pltpu.SemaphoreType.DMA] * 20

    mesh = plsc.VectorSubcoreMesh(core_axis_name="c", subcore_axis_name="s")

    @functools.partial(
        pl.kernel,
        out_type=(jax.ShapeDtypeStruct((HEADS, NP, D), jnp.float32),
                  jax.ShapeDtypeStruct((HEADS, NP), jnp.float32)),
        mesh=mesh,
        scratch_types=scratch,
    )
    def sc_edge(src_hbm, dst_hbm, s_hbm, ft_hbm, out_hbm, den_hbm, *scr):
        it = iter(scr)
        srcsup = [next(it), next(it)]
        dstsup = [next(it), next(it)]
        tmpS = [next(it) for _ in range(4)]
        tmpD = [next(it) for _ in range(4)]
        radj = [next(it) for _ in range(4)]
        sidx = [next(it) for _ in range(4)]
        tsb = [next(it) for _ in range(4)]
        tdb = [next(it) for _ in range(4)]
        exb = [next(it) for _ in range(4)]
        rb = [next(it) for _ in range(4)]
        agg = next(it)
        den = next(it)
        sbuf = next(it)
        ts = [next(it) for _ in range(4)]
        td = [next(it) for _ in range(4)]
        g = [next(it) for _ in range(4)]
        s = [next(it) for _ in range(4)]
        dsem = [next(it) for _ in range(4)]

        c = lax.axis_index("c")
        sid = lax.axis_index("s")
        zeros16 = jnp.zeros((16,), jnp.float32)
        base = sid * STRIPE

        def memset_rb0():
            def row(r, _):
                for t in range(D // 16):
                    rb[0][r, pl.ds(t * 16, 16)] = zeros16
                return 0
            lax.fori_loop(0, EB, row, 0)

        def zero_stripes():
            memset_rb0()
            for q in range(STRIPE // EB):
                pltpu.sync_copy(rb[0], agg.at[pl.ds(base + q * EB, EB)])
                pltpu.sync_copy(rb[0].at[0, pl.ds(0, EB)],
                                den.at[pl.ds(base + q * EB, EB)])

        def gwait(b):
            pltpu.make_async_copy(ft_hbm.at[pl.ds(0, EB)], rb[b], g[b]).wait()

        def swait(b):
            pltpu.make_async_copy(rb[b], agg.at[pl.ds(0, EB)], s[b]).wait()

        def twait(b):
            pltpu.make_async_copy(sbuf.at[pl.ds(0, EB)], tsb[b], ts[b]).wait()
            pltpu.make_async_copy(sbuf.at[pl.ds(0, EB)], tdb[b], td[b]).wait()

        def dwait(b):
            pltpu.make_async_copy(exb[b], den.at[pl.ds(0, EB)], dsem[b]).wait()

        def scale_block(b):
            def grp(gi, _):
                mv = exb[b][pl.ds(gi * 16, 16)]
                for l in range(16):
                    m = _bcast_lane(mv, l)
                    r = gi * 16 + l
                    for t in range(D // 16):
                        rb[b][r, pl.ds(t * 16, 16)] = (
                            rb[b][r, pl.ds(t * 16, 16)] * m)
                return 0
            lax.fori_loop(0, EB // 16, grp, 0)

        def run_pass(k):
            hd = 2 * c + k
            off_s = jnp.int32(k * NP)
            off_d = jnp.int32((2 + k) * NP)
            ft_off = hd * NP if l2 else jnp.int32(0)

            def prep(p, q, b):
                for t in range(EB // 16):
                    sv = srcsup[p][q, pl.ds(t * 16, 16)]
                    dv = dstsup[p][q, pl.ds(t * 16, 16)]
                    tmpS[b][pl.ds(t * 16, 16)] = sv + off_s
                    radj[b][pl.ds(t * 16, 16)] = sv + ft_off
                    tmpD[b][pl.ds(t * 16, 16)] = dv + off_d
                    sidx[b][pl.ds(t * 16, 16)] = dv
                pltpu.async_copy(sbuf.at[tmpS[b]], tsb[b], ts[b])
                pltpu.async_copy(sbuf.at[tmpD[b]], tdb[b], td[b])
                pltpu.async_copy(ft_hbm.at[radj[b]], rb[b], g[b])

            def ex_compute(b):
                twait(b)
                for t in range(EB // 16):
                    x = tsb[b][pl.ds(t * 16, 16)] + tdb[b][pl.ds(t * 16, 16)]
                    x = jnp.where(x >= 0.0, x, x * 0.2)
                    exb[b][pl.ds(t * 16, 16)] = jnp.exp(x)

            def issue_out(b):
                pltpu.async_copy(rb[b], agg.at[sidx[b]], s[b], add=True)
                pltpu.async_copy(exb[b], den.at[sidx[b]], dsem[b], add=True)

            def load_sup(p, S):
                pltpu.sync_copy(src_hbm.at[sid, S], srcsup[p])
                pltpu.sync_copy(dst_hbm.at[sid, S], dstsup[p])

            PREP_SRC = [(0, 3), (1, 0), (1, 1), (1, 2),
                        (1, 3), (0, 0), (0, 1), (0, 2)]

            def body(n, first):
                for pos in range(8):
                    b = pos % 4
                    ex_compute(b)
                    gwait(b)
                    scale_block(b)
                    issue_out(b)
                    if not (first and pos == 0):
                        bprev = (pos - 1) % 4
                        swait(bprev)
                        dwait(bprev)
                    if pos == 1:
                        load_sup(0, jnp.minimum(2 * n + 2, NSUP - 1))
                    if pos == 5:
                        load_sup(1, jnp.minimum(2 * n + 3, NSUP - 1))
                    pp, qq = PREP_SRC[pos]
                    prep(pp, qq, (pos + 3) % 4)

            load_sup(0, jnp.int32(0))
            load_sup(1, jnp.int32(1))
            prep(0, 0, 0)
            prep(0, 1, 1)
            prep(0, 2, 2)
            body(jnp.int32(0), True)

            def iter_body(n, _):
                body(n, False)
                return 0
            lax.fori_loop(1, NITER, iter_body, 0)

            swait(3)
            dwait(3)
            for b in range(3):
                twait(b)
                gwait(b)
            plsc.subcore_barrier()

            pltpu.sync_copy(agg.at[pl.ds(base, STRIPE)],
                            out_hbm.at[hd, pl.ds(base, STRIPE)])
            pltpu.sync_copy(den.at[pl.ds(base, STRIPE)],
                            den_hbm.at[hd, pl.ds(base, STRIPE)])
            if k == 0:
                zero_stripes()
            plsc.subcore_barrier()

        CH = NP // 4
        slot = 0
        for r in range(4):
            for part in range(4):
                t_owner = r * 4 + part
                if l2:
                    srow = jnp.int32(0 if r < 2 else 1)
                else:
                    srow = (2 * c + r) if r < 2 else (4 + 2 * c + (r - 2))

                @pl.when(sid == t_owner)
                def _(r=r, part=part, srow=srow):
                    pltpu.sync_copy(
                        s_hbm.at[pl.ds(srow * NP + part * CH, CH)],
                        sbuf.at[pl.ds(r * NP + part * CH, CH)])
        del slot
        zero_stripes()
        plsc.subcore_barrier()
        run_pass(0)
        run_pass(1)

    return sc_edge


_sc_edge_l1 = _make_sc_edge(1)
_sc_edge_l2 = _make_sc_edge(4)


def kernel(h, edge_index, e_w, snorm_n, W1, W1_self, a1, We_w, We_b, W2, W2_self, a2):
    del e_w, We_w, We_b
    src = edge_index[0]
    dst = edge_index[1]

    src_p = jnp.full((E_PAD,), N, jnp.int32).at[:E].set(src).reshape(
        NSC, NSUP, 4, EB)
    dst_p = jnp.full((E_PAD,), N, jnp.int32).at[:E].set(dst).reshape(
        NSC, NSUP, 4, EB)

    h_t = jnp.zeros((NP, D), jnp.float32).at[:N].set(h)
    sn_t = jnp.zeros((NP,), jnp.float32).at[:N].set(snorm_n[:, 0])

    u1 = jnp.einsum('hij,hj->hi', W1, a1[:, :D])
    v1 = jnp.einsum('hij,hj->hi', W1, a1[:, D:])
    u8_l1 = jnp.concatenate([u1, v1], axis=0)[None]

    S1 = _scalar_tables(h_t[None], u8_l1)

    P, den1 = _sc_edge_l1(src_p, dst_p, S1.reshape(8 * NP), h_t)

    snd1 = jnp.concatenate([sn_t[:, None], den1.T], axis=1)
    snd1 = jnp.pad(snd1, ((0, 0), (0, 3)))
    h1cols = _finish1(P, h_t, snd1, W1, W1_self)

    w2s = W2 @ a2[:DH]
    w2d = W2 @ a2[DH:]
    u8_l2 = jnp.zeros((HEADS, 8, D), jnp.float32)
    u8_l2 = u8_l2.at[:, 0, :].set(w2s.reshape(HEADS, D))
    u8_l2 = u8_l2.at[:, 1, :].set(w2d.reshape(HEADS, D))
    S2 = _scalar_tables(h1cols, u8_l2)

    Q, den2 = _sc_edge_l2(src_p, dst_p, S2.reshape(8 * NP),
                          h1cols.reshape(HEADS * NP, D))

    snd2 = jnp.concatenate([sn_t[:, None], den2[0][:, None]], axis=1)
    snd2 = jnp.pad(snd2, ((0, 0), (0, 6)))
    h2 = _finish2(Q, h1cols, snd2, W2, W2_self)
    return h2[:N]
